# baseline (speedup 1.0000x reference)
import functools

import jax
import jax.numpy as jnp
from jax.experimental import pallas as pl
from jax.experimental.pallas import tpu as pltpu

_VMEM_LIMIT = 48 * 1024 * 1024
_UNROLL = 64


def _gather_add_kernel(seg_smem, x_ref, tbl_ref, o_ref, *, tn):
    step = pl.program_id(0)
    base = step * tn

    def outer(o, _):
        row0 = o * _UNROLL
        for mi in range(_UNROLL):
            r = row0 + mi
            idx = seg_smem[base + r]
            o_ref[r, 0] = x_ref[r, 0] + tbl_ref[idx, 0]
        return ()

    jax.lax.fori_loop(0, tn // _UNROLL, outer, ())


def _pick_tile(n):
    for tn in (1024, 512, 256, 128, 64):
        if n % tn == 0:
            return tn
    return n


def kernel(x, segment_ids, seg_embed):
    L, B, D = x.shape
    N = L * B
    S = seg_embed.shape[0]
    tn = _pick_tile(N)

    x3d = x.reshape(N, 1, D)
    seg1d = segment_ids.reshape(N).astype(jnp.int32)
    tbl3d = seg_embed.reshape(S, 1, D)

    grid_spec = pltpu.PrefetchScalarGridSpec(
        num_scalar_prefetch=1,
        grid=(N // tn,),
        in_specs=[
            pl.BlockSpec((tn, 1, D), lambda i, seg: (i, 0, 0)),
            pl.BlockSpec((S, 1, D), lambda i, seg: (0, 0, 0)),
        ],
        out_specs=pl.BlockSpec((tn, 1, D), lambda i, seg: (i, 0, 0)),
    )
    out3d = pl.pallas_call(
        functools.partial(_gather_add_kernel, tn=tn),
        grid_spec=grid_spec,
        out_shape=jax.ShapeDtypeStruct((N, 1, D), x.dtype),
        compiler_params=pltpu.CompilerParams(
            dimension_semantics=("arbitrary",),
            vmem_limit_bytes=_VMEM_LIMIT),
    )(seg1d, x3d, tbl3d)
    return out3d.reshape(L, B, D)


# speedup vs baseline: 5.0606x; 5.0606x over previous
import jax
import jax.numpy as jnp
from jax.experimental import pallas as pl
from jax.experimental.pallas import tpu as pltpu

_VMEM_LIMIT = 48 * 1024 * 1024


def _seg_add_kernel(seg_ref, x_ref, tbl_ref, o_ref):
    seg2 = seg_ref[...]
    tn = seg2.shape[0]
    s = tbl_ref.shape[0]
    w = jnp.concatenate(
        [jnp.full((1, 128), 256.0, jnp.float32), jnp.ones((1, 128), jnp.float32)],
        axis=0)
    seg_b = jnp.dot(seg2, w,
                    preferred_element_type=jnp.float32)
    iota128 = jax.lax.broadcasted_iota(jnp.int32, (tn, 128), 1).astype(jnp.float32)
    groups = [(iota128 + float(g * 128) == seg_b).astype(jnp.float32)
              for g in range(s // 128)]
    onehot = jnp.concatenate(groups, axis=1)
    emb = jnp.dot(onehot, tbl_ref[...],
                  preferred_element_type=jnp.float32)
    o_ref[...] = x_ref[...] + emb


def _pick_tile(n):
    for tn in (4096, 2048, 1024, 512, 256, 128, 64, 32, 16, 8):
        if n % tn == 0:
            return tn
    return n


def kernel(x, segment_ids, seg_embed):
    L, B, D = x.shape
    N = L * B
    S = seg_embed.shape[0]
    tn = _pick_tile(N)

    x2d = x.reshape(N, D)
    seg = segment_ids.reshape(N).astype(jnp.int32)
    seg2 = jnp.stack([(seg >> 8).astype(jnp.float32),
                      (seg & 255).astype(jnp.float32)], axis=-1)

    out2d = pl.pallas_call(
        _seg_add_kernel,
        out_shape=jax.ShapeDtypeStruct((N, D), x.dtype),
        grid=(N // tn,),
        in_specs=[
            pl.BlockSpec((tn, 2), lambda i: (i, 0)),
            pl.BlockSpec((tn, D), lambda i: (i, 0)),
            pl.BlockSpec((S, D), lambda i: (0, 0)),
        ],
        out_specs=pl.BlockSpec((tn, D), lambda i: (i, 0)),
        compiler_params=pltpu.CompilerParams(
            dimension_semantics=("parallel",),
            vmem_limit_bytes=_VMEM_LIMIT),
    )(seg2, x2d, seg_embed)
    return out2d.reshape(L, B, D)
